# baseline (speedup 1.0000x reference)
import jax
import jax.numpy as jnp
from jax.experimental import pallas as pl
from jax.experimental.pallas import tpu as pltpu


def kernel(inputs, weight):
    *batches, d1, d2 = inputs.shape
    nbatch = 1
    for b in batches:
        nbatch *= b

    block_rows = 1024
    nblocks = d1 // block_rows

    def body(w_ref, o_ref, sem):
        i = pl.program_id(0)
        copies = [
            pltpu.make_async_copy(
                w_ref,
                o_ref.at[b, pl.ds(i * block_rows, block_rows), :],
                sem,
            )
            for b in range(nbatch)
        ]
        for c in copies:
            c.start()
        for c in copies:
            c.wait()

    out = pl.pallas_call(
        body,
        grid=(nblocks,),
        in_specs=[pl.BlockSpec((block_rows, d2), lambda i: (i, 0))],
        out_specs=pl.BlockSpec(memory_space=pl.ANY),
        out_shape=jax.ShapeDtypeStruct((nbatch, d1, d2), weight.dtype),
        scratch_shapes=[pltpu.SemaphoreType.DMA],
    )(weight)

    return out.reshape(tuple(batches) + (d1, d2))


# speedup vs baseline: 1.0244x; 1.0244x over previous
import jax
import jax.numpy as jnp
from jax.experimental import pallas as pl
from jax.experimental.pallas import tpu as pltpu


def kernel(inputs, weight):
    *batches, d1, d2 = inputs.shape
    nbatch = 1
    for b in batches:
        nbatch *= b

    block_rows = 1024
    nblocks = d1 // block_rows

    def body(w_hbm, o_hbm, buf, sem_in, sem_out):
        i = pl.program_id(0)

        def in_copy(j):
            return pltpu.make_async_copy(
                w_hbm.at[pl.ds(j * block_rows, block_rows), :],
                buf.at[j % 3],
                sem_in,
            )

        def out_copies(j):
            return [
                pltpu.make_async_copy(
                    buf.at[j % 3],
                    o_hbm.at[b, pl.ds(j * block_rows, block_rows), :],
                    sem_out,
                )
                for b in range(nbatch)
            ]

        @pl.when(i == 0)
        def _():
            in_copy(0).start()

        @pl.when(i + 1 < nblocks)
        def _():
            in_copy(i + 1).start()

        @pl.when(i > 0)
        def _():
            for c in out_copies(i - 1):
                c.wait()

        in_copy(i).wait()
        for c in out_copies(i):
            c.start()

        @pl.when(i == nblocks - 1)
        def _():
            for c in out_copies(i):
                c.wait()

    out = pl.pallas_call(
        body,
        grid=(nblocks,),
        in_specs=[pl.BlockSpec(memory_space=pl.ANY)],
        out_specs=pl.BlockSpec(memory_space=pl.ANY),
        out_shape=jax.ShapeDtypeStruct((nbatch, d1, d2), weight.dtype),
        scratch_shapes=[
            pltpu.VMEM((3, block_rows, d2), weight.dtype),
            pltpu.SemaphoreType.DMA,
            pltpu.SemaphoreType.DMA,
        ],
    )(weight)

    return out.reshape(tuple(batches) + (d1, d2))


# speedup vs baseline: 1.0716x; 1.0461x over previous
import jax
import jax.numpy as jnp
from jax.experimental import pallas as pl


def kernel(inputs, weight):
    *batches, d1, d2 = inputs.shape
    nbatch = 1
    for b in batches:
        nbatch *= b

    block_rows = 1024
    nblocks = d1 // block_rows

    def body(w_ref, o_ref):
        o_ref[...] = jnp.broadcast_to(w_ref[...][None], (nbatch, block_rows, d2))

    out = pl.pallas_call(
        body,
        grid=(nblocks,),
        in_specs=[pl.BlockSpec((block_rows, d2), lambda i: (i, 0))],
        out_specs=pl.BlockSpec((nbatch, block_rows, d2), lambda i: (0, i, 0)),
        out_shape=jax.ShapeDtypeStruct((nbatch, d1, d2), weight.dtype),
    )(weight)

    return out.reshape(tuple(batches) + (d1, d2))
